# baseline (speedup 1.0000x reference)
import jax
import jax.numpy as jnp
from jax.experimental import pallas as pl
from jax.experimental.pallas import tpu as pltpu

_MAX_SHIFT_BANDS = 2.0


def _band_slice(x, lo, hi):
    n = x.shape[0]
    parts = []
    if lo < 0:
        parts.append(x[:-lo])
        lo = 0
    parts.append(x[lo:min(hi, n)])
    if hi > n:
        parts.append(x[: hi - n])
    return jnp.concatenate(parts, axis=0) if len(parts) > 1 else parts[0]


def _smile_kernel(x_ref, shift_ref, out_ref):
    x = x_ref[0].astype(jnp.bfloat16)
    shift = shift_ref[0]
    Bh, Hblk, W = x.shape

    ci = jax.lax.broadcasted_iota(jnp.int32, shift.shape, 0)
    c = ci.astype(jnp.float32)
    s = jnp.clip(shift, -_MAX_SHIFT_BANDS, _MAX_SHIFT_BANDS)
    shifted = jnp.clip(c + s, 0.0, Bh - 1.0)
    pix = shifted * (float(Bh) / float(Bh - 1)) - 0.5
    pix = jnp.clip(pix, 0.0, Bh - 1.0)

    w8 = []
    for d in range(-3, 4):
        wd = jax.nn.relu(1.0 - jnp.abs(pix - (c + d))).astype(jnp.bfloat16)
        w8.append(jnp.broadcast_to(wd[:, None, :], (Bh, 8, W)))

    for t in range(Bh // 8):
        b0 = 8 * t
        ws = [w[b0:b0 + 8] for w in w8]
        xs = [_band_slice(x, b0 + k - 3, b0 + k + 5) for k in range(7)]
        for hs in range(Hblk // 8):
            h0 = 8 * hs
            acc = ws[0] * xs[0][:, h0:h0 + 8, :]
            for k in range(1, 7):
                acc = acc + ws[k] * xs[k][:, h0:h0 + 8, :]
            out_ref[0, b0:b0 + 8, h0:h0 + 8, :] = acc.astype(jnp.float32)


def kernel(x, wavelength_shift):
    B, Bh, H, W = x.shape
    HBLK = 64
    WBLK = 128
    grid = (B, H // HBLK, W // WBLK)
    return pl.pallas_call(
        _smile_kernel,
        grid=grid,
        in_specs=[
            pl.BlockSpec((1, Bh, HBLK, WBLK), lambda b, h, w: (b, 0, h, w)),
            pl.BlockSpec((1, Bh, WBLK), lambda b, h, w: (b, 0, w)),
        ],
        out_specs=pl.BlockSpec((1, Bh, HBLK, WBLK),
                               lambda b, h, w: (b, 0, h, w)),
        out_shape=jax.ShapeDtypeStruct((B, Bh, H, W), x.dtype),
        compiler_params=pltpu.CompilerParams(
            dimension_semantics=("arbitrary", "arbitrary", "arbitrary")),
    )(x, wavelength_shift)


# speedup vs baseline: 1.0738x; 1.0738x over previous
import jax
import jax.numpy as jnp
from jax.experimental import pallas as pl
from jax.experimental.pallas import tpu as pltpu

_MAX_SHIFT_BANDS = 2.0


def _band_slice(x, lo, hi):
    n = x.shape[0]
    parts = []
    if lo < 0:
        parts.append(x[:-lo])
        lo = 0
    parts.append(x[lo:min(hi, n)])
    if hi > n:
        parts.append(x[: hi - n])
    return jnp.concatenate(parts, axis=0) if len(parts) > 1 else parts[0]


def _smile_kernel(x_ref, shift_ref, out_ref):
    x = x_ref[0].astype(jnp.bfloat16)
    shift = shift_ref[0]
    Bh, Hblk, W = x.shape

    ci = jax.lax.broadcasted_iota(jnp.int32, shift.shape, 0)
    c = ci.astype(jnp.float32)
    s = jnp.clip(shift, -_MAX_SHIFT_BANDS, _MAX_SHIFT_BANDS)
    shifted = jnp.clip(c + s, 0.0, Bh - 1.0)
    pix = shifted * (float(Bh) / float(Bh - 1)) - 0.5
    pix = jnp.clip(pix, 0.0, Bh - 1.0)

    w8 = []
    for d in range(-3, 4):
        wd = jax.nn.relu(1.0 - jnp.abs(pix - (c + d))).astype(jnp.bfloat16)
        w8.append(jnp.broadcast_to(wd[:, None, :], (Bh, 8, W)))

    for hs in range(Hblk // 8):
        h0 = 8 * hs
        xh = x[:, h0:h0 + 8, :]
        for t in range(Bh // 8):
            b0 = 8 * t
            acc = w8[0][b0:b0 + 8] * _band_slice(xh, b0 - 3, b0 + 5)
            for k in range(1, 7):
                acc = acc + w8[k][b0:b0 + 8] * _band_slice(
                    xh, b0 + k - 3, b0 + k + 5)
            out_ref[0, b0:b0 + 8, h0:h0 + 8, :] = acc.astype(jnp.float32)


def kernel(x, wavelength_shift):
    B, Bh, H, W = x.shape
    HBLK = 64
    grid = (B, H // HBLK)
    return pl.pallas_call(
        _smile_kernel,
        grid=grid,
        in_specs=[
            pl.BlockSpec((1, Bh, HBLK, W), lambda b, h: (b, 0, h, 0)),
            pl.BlockSpec((1, Bh, W), lambda b, h: (b, 0, 0)),
        ],
        out_specs=pl.BlockSpec((1, Bh, HBLK, W), lambda b, h: (b, 0, h, 0)),
        out_shape=jax.ShapeDtypeStruct((B, Bh, H, W), x.dtype),
        compiler_params=pltpu.CompilerParams(
            dimension_semantics=("arbitrary", "arbitrary")),
    )(x, wavelength_shift)


# speedup vs baseline: 1.0811x; 1.0068x over previous
import jax
import jax.numpy as jnp
from jax.experimental import pallas as pl
from jax.experimental.pallas import tpu as pltpu

_MAX_SHIFT_BANDS = 2.0


def _band_slice(x, lo, hi):
    n = x.shape[0]
    parts = []
    if lo < 0:
        parts.append(x[:-lo])
        lo = 0
    parts.append(x[lo:min(hi, n)])
    if hi > n:
        parts.append(x[: hi - n])
    return jnp.concatenate(parts, axis=0) if len(parts) > 1 else parts[0]


def _smile_kernel(x_ref, shift_ref, out_ref):
    x = x_ref[0].astype(jnp.bfloat16)
    shift = shift_ref[0]
    Bh, Hblk, W = x.shape

    ci = jax.lax.broadcasted_iota(jnp.int32, shift.shape, 0)
    c = ci.astype(jnp.float32)
    s = jnp.clip(shift, -_MAX_SHIFT_BANDS, _MAX_SHIFT_BANDS)
    shifted = jnp.clip(c + s, 0.0, Bh - 1.0)
    pix = shifted * (float(Bh) / float(Bh - 1)) - 0.5
    pix = jnp.clip(pix, 0.0, Bh - 1.0)

    w8 = []
    for d in range(-3, 4):
        wd = jax.nn.relu(1.0 - jnp.abs(pix - (c + d))).astype(jnp.bfloat16)
        w8.append(jnp.broadcast_to(wd[:, None, :], (Bh, 8, W)))

    for t in range(Bh // 8):
        b0 = 8 * t
        ws = [w[b0:b0 + 8] for w in w8]
        xs = [_band_slice(x, b0 + k - 3, b0 + k + 5) for k in range(7)]
        for hs in range(Hblk // 8):
            h0 = 8 * hs
            acc = ws[0] * xs[0][:, h0:h0 + 8, :]
            for k in range(1, 7):
                acc = acc + ws[k] * xs[k][:, h0:h0 + 8, :]
            out_ref[0, b0:b0 + 8, h0:h0 + 8, :] = acc.astype(jnp.float32)


def kernel(x, wavelength_shift):
    B, Bh, H, W = x.shape
    HBLK = 64
    grid = (B, H // HBLK)
    return pl.pallas_call(
        _smile_kernel,
        grid=grid,
        in_specs=[
            pl.BlockSpec((1, Bh, HBLK, W), lambda b, h: (b, 0, h, 0)),
            pl.BlockSpec((1, Bh, W), lambda b, h: (b, 0, 0)),
        ],
        out_specs=pl.BlockSpec((1, Bh, HBLK, W), lambda b, h: (b, 0, h, 0)),
        out_shape=jax.ShapeDtypeStruct((B, Bh, H, W), x.dtype),
        compiler_params=pltpu.CompilerParams(
            dimension_semantics=("arbitrary", "arbitrary")),
    )(x, wavelength_shift)


# speedup vs baseline: 1.0868x; 1.0053x over previous
import jax
import jax.numpy as jnp
from jax.experimental import pallas as pl
from jax.experimental.pallas import tpu as pltpu

_MAX_SHIFT_BANDS = 2.0


def _band_slice(x, lo, hi):
    n = x.shape[0]
    parts = []
    if lo < 0:
        parts.append(x[:-lo])
        lo = 0
    parts.append(x[lo:min(hi, n)])
    if hi > n:
        parts.append(x[: hi - n])
    return jnp.concatenate(parts, axis=0) if len(parts) > 1 else parts[0]


def _smile_kernel(x_ref, shift_ref, out_ref):
    x = x_ref[0].astype(jnp.bfloat16)
    shift = shift_ref[0]
    Bh, Hblk, W = x.shape

    ci = jax.lax.broadcasted_iota(jnp.int32, shift.shape, 0)
    c = ci.astype(jnp.float32)
    s = jnp.clip(shift, -_MAX_SHIFT_BANDS, _MAX_SHIFT_BANDS)
    shifted = jnp.clip(c + s, 0.0, Bh - 1.0)
    pix = shifted * (float(Bh) / float(Bh - 1)) - 0.5
    pix = jnp.clip(pix, 0.0, Bh - 1.0)

    w8 = []
    for d in range(-3, 4):
        wd = jax.nn.relu(1.0 - jnp.abs(pix - (c + d)))
        w8.append(jnp.broadcast_to(
            wd[:, None, :], (Bh, 8, W)).astype(jnp.bfloat16))

    for t in range(Bh // 8):
        b0 = 8 * t
        ws = [w[b0:b0 + 8] for w in w8]
        xs = [_band_slice(x, b0 + k - 3, b0 + k + 5) for k in range(7)]
        for hs in range(Hblk // 8):
            h0 = 8 * hs
            acc = ws[0] * xs[0][:, h0:h0 + 8, :]
            for k in range(1, 7):
                acc = acc + ws[k] * xs[k][:, h0:h0 + 8, :]
            out_ref[0, b0:b0 + 8, h0:h0 + 8, :] = acc.astype(jnp.float32)


def kernel(x, wavelength_shift):
    B, Bh, H, W = x.shape
    HBLK = 64
    grid = (B, H // HBLK)
    return pl.pallas_call(
        _smile_kernel,
        grid=grid,
        in_specs=[
            pl.BlockSpec((1, Bh, HBLK, W), lambda b, h: (b, 0, h, 0)),
            pl.BlockSpec((1, Bh, W), lambda b, h: (b, 0, 0)),
        ],
        out_specs=pl.BlockSpec((1, Bh, HBLK, W), lambda b, h: (b, 0, h, 0)),
        out_shape=jax.ShapeDtypeStruct((B, Bh, H, W), x.dtype),
        compiler_params=pltpu.CompilerParams(
            dimension_semantics=("arbitrary", "arbitrary")),
    )(x, wavelength_shift)
